# Initial kernel scaffold; baseline (speedup 1.0000x reference)
import jax
import jax.numpy as jnp
from jax.experimental import pallas as pl


def kernel(seq1, seq2, adj, diff_index, diff_weight, sparse, samp_bias1, samp_bias2, W_gcn, b_gcn, prelu_a, W_disc, b_disc):
    raise NotImplementedError("write your pallas kernel here")


# speedup vs baseline: 4.0112x; 4.0112x over previous
import functools

import jax
import jax.numpy as jnp
from jax import lax
from jax.experimental import pallas as pl
from jax.experimental.pallas import tpu as pltpu
from jax.experimental.pallas import tpu_sc as plsc

N = 10000
E = 320000
D = 128
NP = 10240
N_TILES = 16
EP_TILE = E // N_TILES
K = 80
N_CHUNKS = EP_TILE // K
ROWS_PER_TILE = NP // N_TILES


def _bcast_lane(v16, lane):
    idx = jnp.full((16,), lane, dtype=jnp.int32)
    return lax.gather(
        v16, idx[:, None],
        dimension_numbers=lax.GatherDimensionNumbers(
            offset_dims=(), collapsed_slice_dims=(0,), start_index_map=(0,)),
        slice_sizes=(1,),
        mode=lax.GatherScatterMode.PROMISE_IN_BOUNDS)


def _spmm_body(x1_hbm, x2_hbm, row_hbm, col_hbm, w_hbm, out_hbm,
               row_v, col_v, w_v, rows_v, zb, acc, sem):
    cid = lax.axis_index("c")
    sid = lax.axis_index("s")

    z16 = jnp.zeros((16,), jnp.float32)
    for i in range(16):
        for j in range(D // 16):
            zb[i, pl.ds(j * 16, 16)] = z16
    for t in range(ROWS_PER_TILE // 16):
        off = pl.multiple_of(sid * ROWS_PER_TILE + t * 16, 16)
        pltpu.sync_copy(zb, acc.at[pl.ds(off, 16)])
    plsc.subcore_barrier()

    def chunk(it, carry):
        base = pl.multiple_of(sid * EP_TILE + it * K, 16)
        pltpu.sync_copy(row_hbm.at[pl.ds(base, K)], row_v)
        pltpu.sync_copy(col_hbm.at[pl.ds(base, K)], col_v)
        pltpu.sync_copy(w_hbm.at[pl.ds(base, K)], w_v)

        @pl.when(cid == 0)
        def _():
            pltpu.async_copy(x1_hbm.at[col_v], rows_v, sem).wait()

        @pl.when(cid == 1)
        def _():
            pltpu.async_copy(x2_hbm.at[col_v], rows_v, sem).wait()

        for g in range(K // 16):
            w16 = w_v[pl.ds(g * 16, 16)]
            for i in range(16):
                r = g * 16 + i
                wi = _bcast_lane(w16, i)
                for j in range(D // 16):
                    s = pl.ds(j * 16, 16)
                    rows_v[r, s] = rows_v[r, s] * wi

        pltpu.sync_copy(rows_v, acc.at[row_v], add=True)
        return carry

    lax.fori_loop(0, N_CHUNKS, chunk, 0)
    plsc.subcore_barrier()

    for t in range(ROWS_PER_TILE // K):
        off = pl.multiple_of(sid * ROWS_PER_TILE + t * K, 16)
        obase = pl.multiple_of(cid * NP + off, 16)
        pltpu.sync_copy(acc.at[pl.ds(off, K)], rows_v)
        pltpu.sync_copy(rows_v, out_hbm.at[pl.ds(obase, K)])


_spmm = functools.partial(
    pl.kernel,
    mesh=plsc.VectorSubcoreMesh(core_axis_name="c", subcore_axis_name="s"),
    out_type=jax.ShapeDtypeStruct((2 * NP, D), jnp.float32),
    scratch_types=[
        pltpu.VMEM((K,), jnp.int32),
        pltpu.VMEM((K,), jnp.int32),
        pltpu.VMEM((K,), jnp.float32),
        pltpu.VMEM((K, D), jnp.float32),
        pltpu.VMEM((16, D), jnp.float32),
        pltpu.VMEM_SHARED((NP, D), jnp.float32),
        pltpu.SemaphoreType.DMA,
    ],
)(_spmm_body)


def _mm_body(s1_ref, s2_ref, w_ref, o1_ref, o2_ref):
    dn = (((1,), (1,)), ((), ()))
    o1_ref[...] = lax.dot_general(s1_ref[...], w_ref[...], dn,
                                  preferred_element_type=jnp.float32)
    o2_ref[...] = lax.dot_general(s2_ref[...], w_ref[...], dn,
                                  preferred_element_type=jnp.float32)


def _matmul(s1, s2, w):
    blk = 2000
    grid = N // blk
    return pl.pallas_call(
        _mm_body,
        grid=(grid,),
        in_specs=[
            pl.BlockSpec((blk, D), lambda g: (g, 0)),
            pl.BlockSpec((blk, D), lambda g: (g, 0)),
            pl.BlockSpec((D, D), lambda g: (0, 0)),
        ],
        out_specs=[
            pl.BlockSpec((blk, D), lambda g: (g, 0)),
            pl.BlockSpec((blk, D), lambda g: (g, 0)),
        ],
        out_shape=[
            jax.ShapeDtypeStruct((N, D), jnp.float32),
            jax.ShapeDtypeStruct((N, D), jnp.float32),
        ],
    )(s1, s2, w)


def _epi_body(o1_ref, o2_ref, bg_ref, pa_ref, wd_ref, bd_ref,
              sb1_ref, sb2_ref, logits_ref, h_ref):
    a = pa_ref[...]
    o1 = o1_ref[...] + bg_ref[...]
    o2 = o2_ref[...] + bg_ref[...]
    h1 = jnp.where(o1 > 0, o1, a * o1)
    h2 = jnp.where(o2 > 0, o2, a * o2)
    c = jnp.mean(h1, axis=0, keepdims=True)
    c = jax.nn.sigmoid(c)
    dn = (((1,), (1,)), ((), ()))
    v = lax.dot_general(c, wd_ref[...], dn,
                        preferred_element_type=jnp.float32)
    s1 = lax.dot_general(v, h1, dn, preferred_element_type=jnp.float32)
    s2 = lax.dot_general(v, h2, dn, preferred_element_type=jnp.float32)
    logits_ref[0:1, :] = s1 + bd_ref[...] + sb1_ref[...]
    logits_ref[1:2, :] = s2 + bd_ref[...] + sb2_ref[...]
    h_ref[...] = h1


def _epilogue(o1, o2, bg, pa, wd, bd, sb1, sb2):
    return pl.pallas_call(
        _epi_body,
        out_shape=[
            jax.ShapeDtypeStruct((2, N), jnp.float32),
            jax.ShapeDtypeStruct((N, D), jnp.float32),
        ],
    )(o1, o2, bg, pa, wd, bd, sb1, sb2)


def kernel(seq1, seq2, adj, diff_index, diff_weight, sparse,
           samp_bias1, samp_bias2, W_gcn, b_gcn, prelu_a, W_disc, b_disc):
    row = diff_index[0]
    col = diff_index[1]
    x1, x2 = _matmul(seq1[0], seq2[0], W_gcn)
    agg = _spmm(x1, x2, row, col, diff_weight)
    o1 = agg[:N]
    o2 = agg[NP:NP + N]
    l2, h1 = _epilogue(o1, o2, b_gcn.reshape(1, D), prelu_a.reshape(1, 1),
                       W_disc[0], b_disc.reshape(1, 1),
                       samp_bias1, samp_bias2)
    return (l2.reshape(1, 2 * N), h1)


# speedup vs baseline: 9.7260x; 2.4247x over previous
import functools

import jax
import jax.numpy as jnp
from jax import lax
from jax.experimental import pallas as pl
from jax.experimental.pallas import tpu as pltpu
from jax.experimental.pallas import tpu_sc as plsc

N = 10000
E = 320000
D = 128
NP = 10240
N_TILES = 16
EP_TILE = E // N_TILES
K = 80
N_CHUNKS = EP_TILE // K
ROWS_PER_TILE = NP // N_TILES


def _bcast_lane(v16, lane):
    idx = jnp.full((16,), lane, dtype=jnp.int32)
    return lax.gather(
        v16, idx[:, None],
        dimension_numbers=lax.GatherDimensionNumbers(
            offset_dims=(), collapsed_slice_dims=(0,), start_index_map=(0,)),
        slice_sizes=(1,),
        mode=lax.GatherScatterMode.PROMISE_IN_BOUNDS)


def _spmm_body(x1_hbm, x2_hbm, row_hbm, col_hbm, w_hbm, out_hbm,
               g0, g1, s0, s1, col0, col1, row0, row1, w0, w1, rb0, rb1,
               acc, semg0, semg1, sems0, sems1, seml0, seml1):
    cid = lax.axis_index("c")
    sid = lax.axis_index("s")
    ebase = sid * EP_TILE

    def load_idx(chunk, colb, rowb, wb, sem):
        off = pl.multiple_of(ebase + chunk * K, 16)
        pltpu.async_copy(col_hbm.at[pl.ds(off, K)], colb, sem)
        pltpu.async_copy(row_hbm.at[pl.ds(off, K)], rowb, sem)
        pltpu.async_copy(w_hbm.at[pl.ds(off, K)], wb, sem)

    def wait_idx(colb, rowb, wb, sem):
        pltpu.make_async_copy(col_hbm.at[pl.ds(0, K)], colb, sem).wait()
        pltpu.make_async_copy(row_hbm.at[pl.ds(0, K)], rowb, sem).wait()
        pltpu.make_async_copy(w_hbm.at[pl.ds(0, K)], wb, sem).wait()

    def gather(colb, gbuf, sem):
        @pl.when(cid == 0)
        def _():
            pltpu.async_copy(x1_hbm.at[colb], gbuf, sem)

        @pl.when(cid == 1)
        def _():
            pltpu.async_copy(x2_hbm.at[colb], gbuf, sem)

    def wait_gather(gbuf, sem):
        pltpu.make_async_copy(x1_hbm.at[col0], gbuf, sem).wait()

    def scale(gbuf, sbuf, wb, rowb, rb):
        for g in range(K // 16):
            w16 = wb[pl.ds(g * 16, 16)]
            rb[pl.ds(g * 16, 16)] = rowb[pl.ds(g * 16, 16)]
            for i in range(16):
                r = g * 16 + i
                wi = _bcast_lane(w16, i)
                for j in range(D // 16):
                    s = pl.ds(j * 16, 16)
                    sbuf[r, s] = gbuf[r, s] * wi

    def wait_scatter(sbuf, rb, sem):
        pltpu.make_async_copy(sbuf, acc.at[rb], sem).wait()

    load_idx(0, col0, row0, w0, seml0)
    load_idx(1, col1, row1, w1, seml1)

    z16 = jnp.zeros((16,), jnp.float32)
    for i in range(16):
        for j in range(D // 16):
            s0[i, pl.ds(j * 16, 16)] = z16
    for t in range(ROWS_PER_TILE // 16):
        off = pl.multiple_of(sid * ROWS_PER_TILE + t * 16, 16)
        pltpu.sync_copy(s0.at[pl.ds(0, 16)], acc.at[pl.ds(off, 16)])
    plsc.subcore_barrier()

    wait_idx(col0, row0, w0, seml0)
    gather(col0, g0, semg0)

    def pair(it, carry):
        last = N_CHUNKS // 2 - 1
        wait_gather(g0, semg0)
        wait_idx(col1, row1, w1, seml1)
        gather(col1, g1, semg1)

        @pl.when(it > 0)
        def _():
            wait_scatter(s0, rb0, sems0)

        scale(g0, s0, w0, row0, rb0)
        pltpu.async_copy(s0, acc.at[rb0], sems0, add=True)

        @pl.when(it < last)
        def _():
            load_idx(2 * it + 2, col0, row0, w0, seml0)

        wait_gather(g1, semg1)

        @pl.when(it < last)
        def _():
            wait_idx(col0, row0, w0, seml0)
            gather(col0, g0, semg0)

        @pl.when(it > 0)
        def _():
            wait_scatter(s1, rb1, sems1)

        scale(g1, s1, w1, row1, rb1)
        pltpu.async_copy(s1, acc.at[rb1], sems1, add=True)

        @pl.when(it < last)
        def _():
            load_idx(2 * it + 3, col1, row1, w1, seml1)
        return carry

    lax.fori_loop(0, N_CHUNKS // 2, pair, 0)
    wait_scatter(s0, rb0, sems0)
    wait_scatter(s1, rb1, sems1)
    plsc.subcore_barrier()

    for t in range(ROWS_PER_TILE // K):
        off = pl.multiple_of(sid * ROWS_PER_TILE + t * K, 16)
        obase = pl.multiple_of(cid * NP + off, 16)
        pltpu.sync_copy(acc.at[pl.ds(off, K)], g0)
        pltpu.sync_copy(g0, out_hbm.at[pl.ds(obase, K)])


_spmm = functools.partial(
    pl.kernel,
    mesh=plsc.VectorSubcoreMesh(core_axis_name="c", subcore_axis_name="s"),
    out_type=jax.ShapeDtypeStruct((2 * NP, D), jnp.float32),
    scratch_types=[
        pltpu.VMEM((K, D), jnp.float32),
        pltpu.VMEM((K, D), jnp.float32),
        pltpu.VMEM((K, D), jnp.float32),
        pltpu.VMEM((K, D), jnp.float32),
        pltpu.VMEM((K,), jnp.int32),
        pltpu.VMEM((K,), jnp.int32),
        pltpu.VMEM((K,), jnp.int32),
        pltpu.VMEM((K,), jnp.int32),
        pltpu.VMEM((K,), jnp.float32),
        pltpu.VMEM((K,), jnp.float32),
        pltpu.VMEM((K,), jnp.int32),
        pltpu.VMEM((K,), jnp.int32),
        pltpu.VMEM_SHARED((NP, D), jnp.float32),
        pltpu.SemaphoreType.DMA,
        pltpu.SemaphoreType.DMA,
        pltpu.SemaphoreType.DMA,
        pltpu.SemaphoreType.DMA,
        pltpu.SemaphoreType.DMA,
        pltpu.SemaphoreType.DMA,
    ],
)(_spmm_body)


def _mm_body(s1_ref, s2_ref, w_ref, o1_ref, o2_ref):
    dn = (((1,), (1,)), ((), ()))
    o1_ref[...] = lax.dot_general(s1_ref[...], w_ref[...], dn,
                                  preferred_element_type=jnp.float32)
    o2_ref[...] = lax.dot_general(s2_ref[...], w_ref[...], dn,
                                  preferred_element_type=jnp.float32)


def _matmul(s1, s2, w):
    blk = 2000
    grid = N // blk
    return pl.pallas_call(
        _mm_body,
        grid=(grid,),
        in_specs=[
            pl.BlockSpec((blk, D), lambda g: (g, 0)),
            pl.BlockSpec((blk, D), lambda g: (g, 0)),
            pl.BlockSpec((D, D), lambda g: (0, 0)),
        ],
        out_specs=[
            pl.BlockSpec((blk, D), lambda g: (g, 0)),
            pl.BlockSpec((blk, D), lambda g: (g, 0)),
        ],
        out_shape=[
            jax.ShapeDtypeStruct((N, D), jnp.float32),
            jax.ShapeDtypeStruct((N, D), jnp.float32),
        ],
    )(s1, s2, w)


def _epi_body(o1_ref, o2_ref, bg_ref, pa_ref, wd_ref, bd_ref,
              sb1_ref, sb2_ref, logits_ref, h_ref):
    a = pa_ref[...]
    o1 = o1_ref[...] + bg_ref[...]
    o2 = o2_ref[...] + bg_ref[...]
    h1 = jnp.where(o1 > 0, o1, a * o1)
    h2 = jnp.where(o2 > 0, o2, a * o2)
    c = jnp.mean(h1, axis=0, keepdims=True)
    c = jax.nn.sigmoid(c)
    dn = (((1,), (1,)), ((), ()))
    v = lax.dot_general(c, wd_ref[...], dn,
                        preferred_element_type=jnp.float32)
    s1 = lax.dot_general(v, h1, dn, preferred_element_type=jnp.float32)
    s2 = lax.dot_general(v, h2, dn, preferred_element_type=jnp.float32)
    logits_ref[0:1, :] = s1 + bd_ref[...] + sb1_ref[...]
    logits_ref[1:2, :] = s2 + bd_ref[...] + sb2_ref[...]
    h_ref[...] = h1


def _epilogue(o1, o2, bg, pa, wd, bd, sb1, sb2):
    return pl.pallas_call(
        _epi_body,
        out_shape=[
            jax.ShapeDtypeStruct((2, N), jnp.float32),
            jax.ShapeDtypeStruct((N, D), jnp.float32),
        ],
    )(o1, o2, bg, pa, wd, bd, sb1, sb2)


def kernel(seq1, seq2, adj, diff_index, diff_weight, sparse,
           samp_bias1, samp_bias2, W_gcn, b_gcn, prelu_a, W_disc, b_disc):
    row = diff_index[0]
    col = diff_index[1]
    x1, x2 = _matmul(seq1[0], seq2[0], W_gcn)
    agg = _spmm(x1, x2, row, col, diff_weight)
    o1 = agg[:N]
    o2 = agg[NP:NP + N]
    l2, h1 = _epilogue(o1, o2, b_gcn.reshape(1, D), prelu_a.reshape(1, 1),
                       W_disc[0], b_disc.reshape(1, 1),
                       samp_bias1, samp_bias2)
    return (l2.reshape(1, 2 * N), h1)
